# baseline (speedup 1.0000x reference)
import functools

import jax
import jax.numpy as jnp
from jax import lax
from jax.experimental import pallas as pl
from jax.experimental.pallas import tpu as pltpu
from jax.experimental.pallas import tpu_sc as plsc

N = 10000
D = 128
E = 320000

NC = 2
NS = 16
NW = NC * NS
L = 16

CHUNK = 96
T_FAST = 212
T_SLOW = 0
FAST_CORE = 0
W_ST = 80
TOT_CH = NS * (T_FAST + T_SLOW)
T_DEG = TOT_CH // NW
E_PAD = TOT_CH * CHUNK
NPAD = 10240
ROWS_PER_TILE = NPAD // NS
SINK = N + 64

_mesh = functools.partial(
    plsc.VectorSubcoreMesh, core_axis_name="c", subcore_axis_name="s"
)
_sc_params = pltpu.CompilerParams(
    use_tc_tiling_on_sc=False, needs_layout_passes=False
)


@functools.partial(
    pl.kernel,
    out_type=jax.ShapeDtypeStruct((NC, NPAD), jnp.float32),
    mesh=_mesh(),
    compiler_params=_sc_params,
    scratch_types=[
        pltpu.VMEM((T_DEG, CHUNK), jnp.int32),
        pltpu.VMEM((NPAD,), jnp.float32),
        pltpu.VMEM((NS, ROWS_PER_TILE), jnp.float32),
        pltpu.VMEM((ROWS_PER_TILE,), jnp.float32),
        pltpu.VMEM_SHARED((NS, NPAD), jnp.float32),
    ],
)
def _deg_kernel(colf_hbm, out_hbm, colbuf, dloc, partbuf, outbuf, shared):
    c = lax.axis_index("c")
    s = lax.axis_index("s")
    w = c * NS + s

    zeros16 = jnp.zeros((L,), jnp.float32)
    ones16 = jnp.ones((L,), jnp.float32)

    def zero_body(i, _):
        dloc[pl.ds(i * L, L)] = zeros16
        return 0

    lax.fori_loop(0, NPAD // L, zero_body, 0)

    pltpu.sync_copy(colf_hbm.at[pl.ds(w * T_DEG, T_DEG)], colbuf)

    def acc_body(j, _):
        for k in range(CHUNK // L):
            idx = colbuf[j, pl.ds(k * L, L)]
            plsc.addupdate_scatter(dloc, [idx], ones16)
        return 0

    lax.fori_loop(0, T_DEG, acc_body, 0)

    pltpu.sync_copy(dloc, shared.at[s])
    plsc.subcore_barrier()

    def gather_part(p, _):
        pltpu.sync_copy(
            shared.at[p, pl.ds(s * ROWS_PER_TILE, ROWS_PER_TILE)],
            partbuf.at[p],
        )
        return 0

    lax.fori_loop(0, NS, gather_part, 0)

    def sum_body(v, _):
        tot = partbuf[0, pl.ds(v * L, L)]
        for p in range(1, NS):
            tot = tot + partbuf[p, pl.ds(v * L, L)]
        outbuf[pl.ds(v * L, L)] = tot
        return 0

    lax.fori_loop(0, ROWS_PER_TILE // L, sum_body, 0)

    pltpu.sync_copy(
        outbuf, out_hbm.at[c, pl.ds(s * ROWS_PER_TILE, ROWS_PER_TILE)]
    )


@functools.partial(
    pl.kernel,
    out_type=jax.ShapeDtypeStruct((NC, NPAD, D), jnp.float32),
    mesh=_mesh(),
    compiler_params=_sc_params,
    scratch_types=[
        pltpu.VMEM((W_ST, CHUNK), jnp.int32),
        pltpu.VMEM((W_ST, CHUNK), jnp.int32),
        pltpu.VMEM((CHUNK, D), jnp.float32),
        pltpu.VMEM((CHUNK, D), jnp.float32),
        pltpu.VMEM_SHARED((NPAD, D), jnp.float32),
        pltpu.SemaphoreType.DMA,
        pltpu.SemaphoreType.DMA,
    ],
)
def _prop_kernel(zp_hbm, rowf_hbm, colf_hbm, zeros_hbm, out_hbm,
                 ridx_st, cidx_st, rows0, rows1, acc, gsem, ssem):
    c = lax.axis_index("c")
    s = lax.axis_index("s")
    base = s * ROWS_PER_TILE

    pltpu.sync_copy(
        zeros_hbm.at[pl.ds(base, ROWS_PER_TILE)],
        acc.at[pl.ds(base, ROWS_PER_TILE)],
    )
    plsc.subcore_barrier()

    def half(j, rows_this, rows_other, n):
        @pl.when(j >= 1)
        def _():
            pltpu.make_async_copy(
                rows_other, acc.at[cidx_st.at[j - 1]], ssem
            ).wait()

        @pl.when(j + 1 < n)
        def _():
            pltpu.async_copy(zp_hbm.at[ridx_st.at[j + 1]], rows_other, gsem)

        pltpu.make_async_copy(
            zp_hbm.at[ridx_st.at[j]], rows_this, gsem
        ).wait()
        pltpu.async_copy(rows_this, acc.at[cidx_st.at[j]], ssem, add=True)

    def window(sb, n):
        pltpu.sync_copy(rowf_hbm.at[pl.ds(sb, n)], ridx_st.at[pl.ds(0, n)])
        pltpu.sync_copy(colf_hbm.at[pl.ds(sb, n)], cidx_st.at[pl.ds(0, n)])
        pltpu.async_copy(zp_hbm.at[ridx_st.at[0]], rows0, gsem)

        def body(j2, _):
            half(2 * j2, rows0, rows1, n)
            half(2 * j2 + 1, rows1, rows0, n)
            return 0

        lax.fori_loop(0, n // 2, body, 0)
        pltpu.make_async_copy(
            rows1, acc.at[cidx_st.at[n - 1]], ssem
        ).wait()

    @pl.when(c == FAST_CORE)
    def _():
        base_ch = s * T_FAST
        for st in range(T_FAST // W_ST):
            window(base_ch + st * W_ST, W_ST)
        if T_FAST % W_ST:
            window(base_ch + (T_FAST // W_ST) * W_ST, T_FAST % W_ST)

    if T_SLOW:
        @pl.when(c != FAST_CORE)
        def _():
            base_ch = NS * T_FAST + s * T_SLOW
            window(base_ch, T_SLOW)

    plsc.subcore_barrier()
    pltpu.sync_copy(
        acc.at[pl.ds(base, ROWS_PER_TILE)],
        out_hbm.at[c, pl.ds(base, ROWS_PER_TILE)],
    )


def _bn(x, gamma, beta, eps=1e-5):
    mean = jnp.mean(x, axis=0)
    var = jnp.mean(jnp.square(x - mean), axis=0)
    return (x - mean) * lax.rsqrt(var + eps) * gamma + beta


def _tc1_body(x_ref, wg_ref, degp_ref, z1_ref, dis_ref):
    deg = degp_ref[0, :N] + degp_ref[1, :N] + 1.0
    dis = lax.rsqrt(deg)
    dis_ref[...] = dis
    xl = jnp.dot(x_ref[...], wg_ref[...], preferred_element_type=jnp.float32)
    z1_ref[...] = xl * dis[:, None]


def _tc2_body(acc_ref, z1_ref, dis_ref, bg_ref, g1_ref, be1_ref, wd_ref,
              h_ref, z2_ref):
    dis = dis_ref[...]
    z1 = z1_ref[...]
    tot = acc_ref[0, :N, :] + acc_ref[1, :N, :] + z1
    h = _bn(tot * dis[:, None] + bg_ref[...], g1_ref[...], be1_ref[...])
    h_ref[...] = h
    xd = jnp.dot(h, wd_ref[...], preferred_element_type=jnp.float32)
    z2_ref[...] = xd * dis[:, None]


def _tc3_body(acc_ref, z2_ref, h_ref, dis_ref, bd_ref, g3_ref, be3_ref,
              wo_ref, bo_ref, out_ref):
    dis = dis_ref[...]
    h = h_ref[...]
    tot = acc_ref[0, :N, :] + acc_ref[1, :N, :] + z2_ref[...]
    o = tot * dis[:, None] * h + bd_ref[...]
    o = _bn(o, g3_ref[...], be3_ref[...])
    out_ref[...] = (
        jnp.dot(h, wo_ref[:D, :], preferred_element_type=jnp.float32)
        + jnp.dot(o, wo_ref[D:, :], preferred_element_type=jnp.float32)
        + bo_ref[...]
    )


def kernel(x, W_gcn, b_gcn, g1, be1, W_deg, b_deg, g3, be3, W_out, b_out,
           edge_index):
    row = edge_index[0]
    col = edge_index[1]
    pad = E_PAD - E
    rowp = jnp.concatenate([row, jnp.zeros((pad,), jnp.int32)])
    colp = jnp.concatenate([col, jnp.full((pad,), SINK, jnp.int32)])
    row3d = rowp.reshape(TOT_CH, CHUNK)
    col3d = colp.reshape(TOT_CH, CHUNK)
    zeros_acc = jnp.zeros((NPAD, D), jnp.float32)

    deg_partials = _deg_kernel(col3d)

    z1, dis = pl.pallas_call(
        _tc1_body,
        out_shape=(
            jax.ShapeDtypeStruct((N, D), jnp.float32),
            jax.ShapeDtypeStruct((N,), jnp.float32),
        ),
    )(x, W_gcn, deg_partials)

    acc1 = _prop_kernel(z1, row3d, col3d, zeros_acc)

    h, z2 = pl.pallas_call(
        _tc2_body,
        out_shape=(
            jax.ShapeDtypeStruct((N, D), jnp.float32),
            jax.ShapeDtypeStruct((N, D), jnp.float32),
        ),
    )(acc1, z1, dis, b_gcn, g1, be1, W_deg)

    acc2 = _prop_kernel(z2, row3d, col3d, zeros_acc)

    out = pl.pallas_call(
        _tc3_body,
        out_shape=jax.ShapeDtypeStruct((N, D), jnp.float32),
    )(acc2, z2, h, dis, b_deg, g3, be3, W_out, b_out)

    return out


# speedup vs baseline: 1.1028x; 1.1028x over previous
import functools

import jax
import jax.numpy as jnp
from jax import lax
from jax.experimental import pallas as pl
from jax.experimental.pallas import tpu as pltpu
from jax.experimental.pallas import tpu_sc as plsc

N = 10000
D = 128
E = 320000

NC = 2
NS = 16
NW = NC * NS
L = 16

CHUNK = 96
T_FAST = 172
T_SLOW = 40
FAST_CORE = 0
W_ST = 86
TOT_CH = NS * (T_FAST + T_SLOW)
T_DEG = TOT_CH // NW
E_PAD = TOT_CH * CHUNK
NPAD = 10240
ROWS_PER_TILE = NPAD // NS
SINK = N + 64

_mesh = functools.partial(
    plsc.VectorSubcoreMesh, core_axis_name="c", subcore_axis_name="s"
)
_sc_params = pltpu.CompilerParams(
    use_tc_tiling_on_sc=False, needs_layout_passes=False
)


@functools.partial(
    pl.kernel,
    out_type=jax.ShapeDtypeStruct((NC, NPAD), jnp.float32),
    mesh=_mesh(),
    compiler_params=_sc_params,
    scratch_types=[
        pltpu.VMEM((T_DEG, CHUNK), jnp.int32),
        pltpu.VMEM((NPAD,), jnp.float32),
        pltpu.VMEM((NS, ROWS_PER_TILE), jnp.float32),
        pltpu.VMEM((ROWS_PER_TILE,), jnp.float32),
        pltpu.VMEM_SHARED((NS, NPAD), jnp.float32),
    ],
)
def _deg_kernel(colf_hbm, out_hbm, colbuf, dloc, partbuf, outbuf, shared):
    c = lax.axis_index("c")
    s = lax.axis_index("s")
    w = c * NS + s

    zeros16 = jnp.zeros((L,), jnp.float32)
    ones16 = jnp.ones((L,), jnp.float32)

    def zero_body(i, _):
        dloc[pl.ds(i * L, L)] = zeros16
        return 0

    lax.fori_loop(0, NPAD // L, zero_body, 0)

    pltpu.sync_copy(colf_hbm.at[pl.ds(w * T_DEG, T_DEG)], colbuf)

    def acc_body(j, _):
        for k in range(CHUNK // L):
            idx = colbuf[j, pl.ds(k * L, L)]
            plsc.addupdate_scatter(dloc, [idx], ones16)
        return 0

    lax.fori_loop(0, T_DEG, acc_body, 0)

    pltpu.sync_copy(dloc, shared.at[s])
    plsc.subcore_barrier()

    def gather_part(p, _):
        pltpu.sync_copy(
            shared.at[p, pl.ds(s * ROWS_PER_TILE, ROWS_PER_TILE)],
            partbuf.at[p],
        )
        return 0

    lax.fori_loop(0, NS, gather_part, 0)

    def sum_body(v, _):
        tot = partbuf[0, pl.ds(v * L, L)]
        for p in range(1, NS):
            tot = tot + partbuf[p, pl.ds(v * L, L)]
        outbuf[pl.ds(v * L, L)] = tot
        return 0

    lax.fori_loop(0, ROWS_PER_TILE // L, sum_body, 0)

    pltpu.sync_copy(
        outbuf, out_hbm.at[c, pl.ds(s * ROWS_PER_TILE, ROWS_PER_TILE)]
    )


@functools.partial(
    pl.kernel,
    out_type=jax.ShapeDtypeStruct((NC, NPAD, D), jnp.float32),
    mesh=_mesh(),
    compiler_params=_sc_params,
    scratch_types=[
        pltpu.VMEM((W_ST, CHUNK), jnp.int32),
        pltpu.VMEM((W_ST, CHUNK), jnp.int32),
        pltpu.VMEM((CHUNK, D // 2), jnp.uint32),
        pltpu.VMEM((CHUNK, D // 2), jnp.uint32),
        pltpu.VMEM((CHUNK, D), jnp.float32),
        pltpu.VMEM_SHARED((NPAD, D), jnp.float32),
        pltpu.SemaphoreType.DMA,
        pltpu.SemaphoreType.DMA,
    ],
)
def _prop_kernel(zp_hbm, rowf_hbm, colf_hbm, zeros_hbm, out_hbm,
                 ridx_st, cidx_st, bf0, bf1, fbuf, acc, gsem, ssem):
    c = lax.axis_index("c")
    s = lax.axis_index("s")
    base = s * ROWS_PER_TILE

    pltpu.sync_copy(
        zeros_hbm.at[pl.ds(base, ROWS_PER_TILE)],
        acc.at[pl.ds(base, ROWS_PER_TILE)],
    )
    plsc.subcore_barrier()

    def convert(bf_src):
        def crow(k, _):
            for g in range(D // 32):
                v = bf_src[k, pl.ds(16 * g, 16)]
                vb = plsc.bitcast(v, jnp.bfloat16)
                a, b = plsc.unpack(vb, format=plsc.PackFormat.INTERLEAVED)
                fbuf[k, pl.ds(32 * g, 16)] = a
                fbuf[k, pl.ds(32 * g + 16, 16)] = b
            return 0

        lax.fori_loop(0, CHUNK, crow, 0)

    def half(j, bf_this, bf_other, n):
        pltpu.make_async_copy(
            zp_hbm.at[ridx_st.at[j]], bf_this, gsem
        ).wait()

        @pl.when(j + 1 < n)
        def _():
            pltpu.async_copy(zp_hbm.at[ridx_st.at[j + 1]], bf_other, gsem)

        @pl.when(j >= 1)
        def _():
            pltpu.make_async_copy(
                fbuf, acc.at[cidx_st.at[j - 1]], ssem
            ).wait()

        convert(bf_this)
        pltpu.async_copy(fbuf, acc.at[cidx_st.at[j]], ssem, add=True)

    def window(sb, n):
        pltpu.sync_copy(rowf_hbm.at[pl.ds(sb, n)], ridx_st.at[pl.ds(0, n)])
        pltpu.sync_copy(colf_hbm.at[pl.ds(sb, n)], cidx_st.at[pl.ds(0, n)])
        pltpu.async_copy(zp_hbm.at[ridx_st.at[0]], bf0, gsem)

        def body(j2, _):
            half(2 * j2, bf0, bf1, n)
            half(2 * j2 + 1, bf1, bf0, n)
            return 0

        lax.fori_loop(0, n // 2, body, 0)
        pltpu.make_async_copy(
            fbuf, acc.at[cidx_st.at[n - 1]], ssem
        ).wait()

    @pl.when(c == FAST_CORE)
    def _():
        base_ch = s * T_FAST
        for st in range(T_FAST // W_ST):
            window(base_ch + st * W_ST, W_ST)

    @pl.when(c != FAST_CORE)
    def _():
        base_ch = NS * T_FAST + s * T_SLOW
        window(base_ch, T_SLOW)

    plsc.subcore_barrier()
    pltpu.sync_copy(
        acc.at[pl.ds(base, ROWS_PER_TILE)],
        out_hbm.at[c, pl.ds(base, ROWS_PER_TILE)],
    )


def _bn(x, gamma, beta, eps=1e-5):
    mean = jnp.mean(x, axis=0)
    var = jnp.mean(jnp.square(x - mean), axis=0)
    return (x - mean) * lax.rsqrt(var + eps) * gamma + beta


def _pack_bf16_words(z):
    lo = jnp.concatenate([z[:, 32 * g:32 * g + 16] for g in range(D // 32)],
                         axis=1)
    hi = jnp.concatenate([z[:, 32 * g + 16:32 * g + 32] for g in range(D // 32)],
                         axis=1)
    lw = lax.bitcast_convert_type(lo.astype(jnp.bfloat16), jnp.uint16)
    hw = lax.bitcast_convert_type(hi.astype(jnp.bfloat16), jnp.uint16)
    return lw.astype(jnp.uint32) | (hw.astype(jnp.uint32) << 16)


def _tc1_body(x_ref, wg_ref, degp_ref, z1_ref, z1b_ref, dis_ref):
    deg = degp_ref[0, :N] + degp_ref[1, :N] + 1.0
    dis = lax.rsqrt(deg)
    dis_ref[...] = dis
    xl = jnp.dot(x_ref[...], wg_ref[...], preferred_element_type=jnp.float32)
    z1 = xl * dis[:, None]
    z1_ref[...] = z1
    z1b_ref[...] = _pack_bf16_words(z1)


def _tc2_body(acc_ref, z1_ref, dis_ref, bg_ref, g1_ref, be1_ref, wd_ref,
              h_ref, z2_ref, z2b_ref):
    dis = dis_ref[...]
    z1 = z1_ref[...]
    tot = acc_ref[0, :N, :] + acc_ref[1, :N, :] + z1
    h = _bn(tot * dis[:, None] + bg_ref[...], g1_ref[...], be1_ref[...])
    h_ref[...] = h
    xd = jnp.dot(h, wd_ref[...], preferred_element_type=jnp.float32)
    z2 = xd * dis[:, None]
    z2_ref[...] = z2
    z2b_ref[...] = _pack_bf16_words(z2)


def _tc3_body(acc_ref, z2_ref, h_ref, dis_ref, bd_ref, g3_ref, be3_ref,
              wo_ref, bo_ref, out_ref):
    dis = dis_ref[...]
    h = h_ref[...]
    tot = acc_ref[0, :N, :] + acc_ref[1, :N, :] + z2_ref[...]
    o = tot * dis[:, None] * h + bd_ref[...]
    o = _bn(o, g3_ref[...], be3_ref[...])
    out_ref[...] = (
        jnp.dot(h, wo_ref[:D, :], preferred_element_type=jnp.float32)
        + jnp.dot(o, wo_ref[D:, :], preferred_element_type=jnp.float32)
        + bo_ref[...]
    )


def kernel(x, W_gcn, b_gcn, g1, be1, W_deg, b_deg, g3, be3, W_out, b_out,
           edge_index):
    row = edge_index[0]
    col = edge_index[1]
    pad = E_PAD - E
    rowp = jnp.concatenate([row, jnp.zeros((pad,), jnp.int32)])
    colp = jnp.concatenate([col, jnp.full((pad,), SINK, jnp.int32)])
    row3d = rowp.reshape(TOT_CH, CHUNK)
    col3d = colp.reshape(TOT_CH, CHUNK)
    zeros_acc = jnp.zeros((NPAD, D), jnp.float32)

    deg_partials = _deg_kernel(col3d)

    z1, z1b, dis = pl.pallas_call(
        _tc1_body,
        out_shape=(
            jax.ShapeDtypeStruct((N, D), jnp.float32),
            jax.ShapeDtypeStruct((N, D // 2), jnp.uint32),
            jax.ShapeDtypeStruct((N,), jnp.float32),
        ),
    )(x, W_gcn, deg_partials)

    acc1 = _prop_kernel(z1b, row3d, col3d, zeros_acc)

    h, z2, z2b = pl.pallas_call(
        _tc2_body,
        out_shape=(
            jax.ShapeDtypeStruct((N, D), jnp.float32),
            jax.ShapeDtypeStruct((N, D), jnp.float32),
            jax.ShapeDtypeStruct((N, D // 2), jnp.uint32),
        ),
    )(acc1, z1, dis, b_gcn, g1, be1, W_deg)

    acc2 = _prop_kernel(z2b, row3d, col3d, zeros_acc)

    out = pl.pallas_call(
        _tc3_body,
        out_shape=jax.ShapeDtypeStruct((N, D), jnp.float32),
    )(acc2, z2, h, dis, b_deg, g3, be3, W_out, b_out)

    return out


# speedup vs baseline: 1.1030x; 1.0002x over previous
import functools

import jax
import jax.numpy as jnp
from jax import lax
from jax.experimental import pallas as pl
from jax.experimental.pallas import tpu as pltpu
from jax.experimental.pallas import tpu_sc as plsc

N = 10000
D = 128
E = 320000

NC = 2
NS = 16
NW = NC * NS
L = 16

CHUNK = 96
T_FAST = 172
T_SLOW = 40
FAST_CORE = 0
W_ST = 86
TOT_CH = NS * (T_FAST + T_SLOW)
T_DEG = TOT_CH // NW
E_PAD = TOT_CH * CHUNK
NPAD = 10240
ROWS_PER_TILE = NPAD // NS
SINK = N + 64

_mesh = functools.partial(
    plsc.VectorSubcoreMesh, core_axis_name="c", subcore_axis_name="s"
)
_sc_params = pltpu.CompilerParams(
    use_tc_tiling_on_sc=False, needs_layout_passes=False
)


@functools.partial(
    pl.kernel,
    out_type=jax.ShapeDtypeStruct((NC, NPAD), jnp.float32),
    mesh=_mesh(),
    compiler_params=_sc_params,
    scratch_types=[
        pltpu.VMEM((T_DEG, CHUNK), jnp.int32),
        pltpu.VMEM((NPAD,), jnp.float32),
        pltpu.VMEM((NS, ROWS_PER_TILE), jnp.float32),
        pltpu.VMEM((ROWS_PER_TILE,), jnp.float32),
        pltpu.VMEM_SHARED((NS, NPAD), jnp.float32),
    ],
)
def _deg_kernel(colf_hbm, out_hbm, colbuf, dloc, partbuf, outbuf, shared):
    c = lax.axis_index("c")
    s = lax.axis_index("s")
    w = c * NS + s

    zeros16 = jnp.zeros((L,), jnp.float32)
    ones16 = jnp.ones((L,), jnp.float32)

    def zero_body(i, _):
        dloc[pl.ds(i * L, L)] = zeros16
        return 0

    lax.fori_loop(0, NPAD // L, zero_body, 0)

    pltpu.sync_copy(colf_hbm.at[pl.ds(w * T_DEG, T_DEG)], colbuf)

    def acc_body(j, _):
        for k in range(CHUNK // L):
            idx = colbuf[j, pl.ds(k * L, L)]
            plsc.addupdate_scatter(dloc, [idx], ones16)
        return 0

    lax.fori_loop(0, T_DEG, acc_body, 0)

    pltpu.sync_copy(dloc, shared.at[s])
    plsc.subcore_barrier()

    def gather_part(p, _):
        pltpu.sync_copy(
            shared.at[p, pl.ds(s * ROWS_PER_TILE, ROWS_PER_TILE)],
            partbuf.at[p],
        )
        return 0

    lax.fori_loop(0, NS, gather_part, 0)

    def sum_body(v, _):
        tot = partbuf[0, pl.ds(v * L, L)]
        for p in range(1, NS):
            tot = tot + partbuf[p, pl.ds(v * L, L)]
        outbuf[pl.ds(v * L, L)] = tot
        return 0

    lax.fori_loop(0, ROWS_PER_TILE // L, sum_body, 0)

    pltpu.sync_copy(
        outbuf, out_hbm.at[c, pl.ds(s * ROWS_PER_TILE, ROWS_PER_TILE)]
    )


@functools.partial(
    pl.kernel,
    out_type=jax.ShapeDtypeStruct((NC, NPAD, D), jnp.float32),
    mesh=_mesh(),
    compiler_params=_sc_params,
    scratch_types=[
        pltpu.VMEM((W_ST, CHUNK), jnp.int32),
        pltpu.VMEM((W_ST, CHUNK), jnp.int32),
        pltpu.VMEM((CHUNK, D // 2), jnp.uint32),
        pltpu.VMEM((CHUNK, D // 2), jnp.uint32),
        pltpu.VMEM((CHUNK, D), jnp.float32),
        pltpu.VMEM_SHARED((NPAD, D), jnp.float32),
        pltpu.SemaphoreType.DMA,
        pltpu.SemaphoreType.DMA,
    ],
)
def _prop_kernel(zp_hbm, rowf_hbm, colf_hbm, zeros_hbm, out_hbm,
                 ridx_st, cidx_st, bf0, bf1, fbuf, acc, gsem, ssem):
    c = lax.axis_index("c")
    s = lax.axis_index("s")
    base = s * ROWS_PER_TILE

    pltpu.sync_copy(
        zeros_hbm.at[pl.ds(base, ROWS_PER_TILE)],
        acc.at[pl.ds(base, ROWS_PER_TILE)],
    )
    plsc.subcore_barrier()

    def convert(bf_src):
        hi_mask = jnp.uint32(0xFFFF0000)
        sh = jnp.uint32(16)

        def crow(k, _):
            for g in range(D // 32):
                v = bf_src[k, pl.ds(16 * g, 16)]
                a = plsc.bitcast(v << sh, jnp.float32)
                b = plsc.bitcast(v & hi_mask, jnp.float32)
                fbuf[k, pl.ds(32 * g, 16)] = a
                fbuf[k, pl.ds(32 * g + 16, 16)] = b
            return 0

        lax.fori_loop(0, CHUNK, crow, 0)

    def half(j, bf_this, bf_other, n):
        pltpu.make_async_copy(
            zp_hbm.at[ridx_st.at[j]], bf_this, gsem
        ).wait()

        @pl.when(j + 1 < n)
        def _():
            pltpu.async_copy(zp_hbm.at[ridx_st.at[j + 1]], bf_other, gsem)

        @pl.when(j >= 1)
        def _():
            pltpu.make_async_copy(
                fbuf, acc.at[cidx_st.at[j - 1]], ssem
            ).wait()

        convert(bf_this)
        pltpu.async_copy(fbuf, acc.at[cidx_st.at[j]], ssem, add=True)

    def window(sb, n):
        pltpu.sync_copy(rowf_hbm.at[pl.ds(sb, n)], ridx_st.at[pl.ds(0, n)])
        pltpu.sync_copy(colf_hbm.at[pl.ds(sb, n)], cidx_st.at[pl.ds(0, n)])
        pltpu.async_copy(zp_hbm.at[ridx_st.at[0]], bf0, gsem)

        def body(j2, _):
            half(2 * j2, bf0, bf1, n)
            half(2 * j2 + 1, bf1, bf0, n)
            return 0

        lax.fori_loop(0, n // 2, body, 0)
        pltpu.make_async_copy(
            fbuf, acc.at[cidx_st.at[n - 1]], ssem
        ).wait()

    @pl.when(c == FAST_CORE)
    def _():
        base_ch = s * T_FAST
        for st in range(T_FAST // W_ST):
            window(base_ch + st * W_ST, W_ST)

    @pl.when(c != FAST_CORE)
    def _():
        base_ch = NS * T_FAST + s * T_SLOW
        window(base_ch, T_SLOW)

    plsc.subcore_barrier()
    pltpu.sync_copy(
        acc.at[pl.ds(base, ROWS_PER_TILE)],
        out_hbm.at[c, pl.ds(base, ROWS_PER_TILE)],
    )


def _bn(x, gamma, beta, eps=1e-5):
    mean = jnp.mean(x, axis=0)
    var = jnp.mean(jnp.square(x - mean), axis=0)
    return (x - mean) * lax.rsqrt(var + eps) * gamma + beta


def _pack_bf16_words(z):
    lo = jnp.concatenate([z[:, 32 * g:32 * g + 16] for g in range(D // 32)],
                         axis=1)
    hi = jnp.concatenate([z[:, 32 * g + 16:32 * g + 32] for g in range(D // 32)],
                         axis=1)
    lw = lax.bitcast_convert_type(lo.astype(jnp.bfloat16), jnp.uint16)
    hw = lax.bitcast_convert_type(hi.astype(jnp.bfloat16), jnp.uint16)
    return lw.astype(jnp.uint32) | (hw.astype(jnp.uint32) << 16)


def _tc1_body(x_ref, wg_ref, degp_ref, z1_ref, z1b_ref, dis_ref):
    deg = degp_ref[0, :N] + degp_ref[1, :N] + 1.0
    dis = lax.rsqrt(deg)
    dis_ref[...] = dis
    xl = jnp.dot(x_ref[...], wg_ref[...], preferred_element_type=jnp.float32)
    z1 = xl * dis[:, None]
    z1_ref[...] = z1
    z1b_ref[...] = _pack_bf16_words(z1)


def _tc2_body(acc_ref, z1_ref, dis_ref, bg_ref, g1_ref, be1_ref, wd_ref,
              h_ref, z2_ref, z2b_ref):
    dis = dis_ref[...]
    z1 = z1_ref[...]
    tot = acc_ref[0, :N, :] + acc_ref[1, :N, :] + z1
    h = _bn(tot * dis[:, None] + bg_ref[...], g1_ref[...], be1_ref[...])
    h_ref[...] = h
    xd = jnp.dot(h, wd_ref[...], preferred_element_type=jnp.float32)
    z2 = xd * dis[:, None]
    z2_ref[...] = z2
    z2b_ref[...] = _pack_bf16_words(z2)


def _tc3_body(acc_ref, z2_ref, h_ref, dis_ref, bd_ref, g3_ref, be3_ref,
              wo_ref, bo_ref, out_ref):
    dis = dis_ref[...]
    h = h_ref[...]
    tot = acc_ref[0, :N, :] + acc_ref[1, :N, :] + z2_ref[...]
    o = tot * dis[:, None] * h + bd_ref[...]
    o = _bn(o, g3_ref[...], be3_ref[...])
    out_ref[...] = (
        jnp.dot(h, wo_ref[:D, :], preferred_element_type=jnp.float32)
        + jnp.dot(o, wo_ref[D:, :], preferred_element_type=jnp.float32)
        + bo_ref[...]
    )


def kernel(x, W_gcn, b_gcn, g1, be1, W_deg, b_deg, g3, be3, W_out, b_out,
           edge_index):
    row = edge_index[0]
    col = edge_index[1]
    pad = E_PAD - E
    rowp = jnp.concatenate([row, jnp.zeros((pad,), jnp.int32)])
    colp = jnp.concatenate([col, jnp.full((pad,), SINK, jnp.int32)])
    row3d = rowp.reshape(TOT_CH, CHUNK)
    col3d = colp.reshape(TOT_CH, CHUNK)
    zeros_acc = jnp.zeros((NPAD, D), jnp.float32)

    deg_partials = _deg_kernel(col3d)

    z1, z1b, dis = pl.pallas_call(
        _tc1_body,
        out_shape=(
            jax.ShapeDtypeStruct((N, D), jnp.float32),
            jax.ShapeDtypeStruct((N, D // 2), jnp.uint32),
            jax.ShapeDtypeStruct((N,), jnp.float32),
        ),
    )(x, W_gcn, deg_partials)

    acc1 = _prop_kernel(z1b, row3d, col3d, zeros_acc)

    h, z2, z2b = pl.pallas_call(
        _tc2_body,
        out_shape=(
            jax.ShapeDtypeStruct((N, D), jnp.float32),
            jax.ShapeDtypeStruct((N, D), jnp.float32),
            jax.ShapeDtypeStruct((N, D // 2), jnp.uint32),
        ),
    )(acc1, z1, dis, b_gcn, g1, be1, W_deg)

    acc2 = _prop_kernel(z2b, row3d, col3d, zeros_acc)

    out = pl.pallas_call(
        _tc3_body,
        out_shape=jax.ShapeDtypeStruct((N, D), jnp.float32),
    )(acc2, z2, h, dis, b_deg, g3, be3, W_out, b_out)

    return out


# speedup vs baseline: 2.0112x; 1.8234x over previous
import functools

import jax
import jax.numpy as jnp
from jax import lax
from jax.experimental import pallas as pl
from jax.experimental.pallas import tpu as pltpu
from jax.experimental.pallas import tpu_sc as plsc

N = 10000
D = 128
E = 320000

NC = 2
NS = 16
NW = NC * NS
L = 16

CHUNK = 96
T_FAST = 172
T_SLOW = 40
FAST_CORE = 0
W_ST = 86
TOT_CH = NS * (T_FAST + T_SLOW)
T_DEG = TOT_CH // NW
E_PAD = TOT_CH * CHUNK
NPAD = 10240
ROWS_PER_TILE = NPAD // NS
SINK = N + 64

_mesh = functools.partial(
    plsc.VectorSubcoreMesh, core_axis_name="c", subcore_axis_name="s"
)
_sc_params = pltpu.CompilerParams(
    use_tc_tiling_on_sc=False, needs_layout_passes=False
)


@functools.partial(
    pl.kernel,
    out_type=jax.ShapeDtypeStruct((NC, NPAD), jnp.float32),
    mesh=_mesh(),
    compiler_params=_sc_params,
    scratch_types=[
        pltpu.VMEM((T_DEG, CHUNK), jnp.int32),
        pltpu.VMEM((NPAD,), jnp.float32),
        pltpu.VMEM((NS, ROWS_PER_TILE), jnp.float32),
        pltpu.VMEM((ROWS_PER_TILE,), jnp.float32),
        pltpu.VMEM_SHARED((NS, NPAD), jnp.float32),
    ],
)
def _deg_kernel(colf_hbm, out_hbm, colbuf, dloc, partbuf, outbuf, shared):
    c = lax.axis_index("c")
    s = lax.axis_index("s")
    w = c * NS + s

    zeros16 = jnp.zeros((L,), jnp.float32)
    ones16 = jnp.ones((L,), jnp.float32)

    def zero_body(i, _):
        dloc[pl.ds(i * L, L)] = zeros16
        return 0

    lax.fori_loop(0, NPAD // L, zero_body, 0)

    pltpu.sync_copy(colf_hbm.at[pl.ds(w * T_DEG, T_DEG)], colbuf)

    def acc_body(j, _):
        for k in range(CHUNK // L):
            idx = colbuf[j, pl.ds(k * L, L)]
            plsc.addupdate_scatter(dloc, [idx], ones16)
        return 0

    lax.fori_loop(0, T_DEG, acc_body, 0)

    pltpu.sync_copy(dloc, shared.at[s])
    plsc.subcore_barrier()

    def gather_part(p, _):
        pltpu.sync_copy(
            shared.at[p, pl.ds(s * ROWS_PER_TILE, ROWS_PER_TILE)],
            partbuf.at[p],
        )
        return 0

    lax.fori_loop(0, NS, gather_part, 0)

    def sum_body(v, _):
        tot = partbuf[0, pl.ds(v * L, L)]
        for p in range(1, NS):
            tot = tot + partbuf[p, pl.ds(v * L, L)]
        outbuf[pl.ds(v * L, L)] = tot
        return 0

    lax.fori_loop(0, ROWS_PER_TILE // L, sum_body, 0)

    pltpu.sync_copy(
        outbuf, out_hbm.at[c, pl.ds(s * ROWS_PER_TILE, ROWS_PER_TILE)]
    )


@functools.partial(
    pl.kernel,
    out_type=jax.ShapeDtypeStruct((NC, NPAD, D), jnp.bfloat16),
    mesh=_mesh(),
    compiler_params=_sc_params,
    scratch_types=[
        pltpu.VMEM((W_ST, CHUNK), jnp.int32),
        pltpu.VMEM((W_ST, CHUNK), jnp.int32),
        pltpu.VMEM((CHUNK, D), jnp.bfloat16),
        pltpu.VMEM((CHUNK, D), jnp.bfloat16),
        pltpu.VMEM_SHARED((NPAD, D), jnp.bfloat16),
        pltpu.SemaphoreType.DMA,
        pltpu.SemaphoreType.DMA,
    ],
)
def _prop_kernel(zp_hbm, rowf_hbm, colf_hbm, zeros_hbm, out_hbm,
                 ridx_st, cidx_st, rows0, rows1, acc, gsem, ssem):
    c = lax.axis_index("c")
    s = lax.axis_index("s")
    base = s * ROWS_PER_TILE

    pltpu.sync_copy(
        zeros_hbm.at[pl.ds(base, ROWS_PER_TILE)],
        acc.at[pl.ds(base, ROWS_PER_TILE)],
    )
    plsc.subcore_barrier()

    def half(j, rows_this, rows_other, n):
        @pl.when(j >= 1)
        def _():
            pltpu.make_async_copy(
                rows_other, acc.at[cidx_st.at[j - 1]], ssem
            ).wait()

        @pl.when(j + 1 < n)
        def _():
            pltpu.async_copy(zp_hbm.at[ridx_st.at[j + 1]], rows_other, gsem)

        pltpu.make_async_copy(
            zp_hbm.at[ridx_st.at[j]], rows_this, gsem
        ).wait()
        pltpu.async_copy(rows_this, acc.at[cidx_st.at[j]], ssem, add=True)

    def window(sb, n):
        pltpu.sync_copy(rowf_hbm.at[pl.ds(sb, n)], ridx_st.at[pl.ds(0, n)])
        pltpu.sync_copy(colf_hbm.at[pl.ds(sb, n)], cidx_st.at[pl.ds(0, n)])
        pltpu.async_copy(zp_hbm.at[ridx_st.at[0]], rows0, gsem)

        def body(j2, _):
            half(2 * j2, rows0, rows1, n)
            half(2 * j2 + 1, rows1, rows0, n)
            return 0

        lax.fori_loop(0, n // 2, body, 0)
        pltpu.make_async_copy(
            rows1, acc.at[cidx_st.at[n - 1]], ssem
        ).wait()

    @pl.when(c == FAST_CORE)
    def _():
        base_ch = s * T_FAST
        for st in range(T_FAST // W_ST):
            window(base_ch + st * W_ST, W_ST)

    @pl.when(c != FAST_CORE)
    def _():
        base_ch = NS * T_FAST + s * T_SLOW
        window(base_ch, T_SLOW)

    plsc.subcore_barrier()
    pltpu.sync_copy(
        acc.at[pl.ds(base, ROWS_PER_TILE)],
        out_hbm.at[c, pl.ds(base, ROWS_PER_TILE)],
    )


def _bn(x, gamma, beta, eps=1e-5):
    mean = jnp.mean(x, axis=0)
    var = jnp.mean(jnp.square(x - mean), axis=0)
    return (x - mean) * lax.rsqrt(var + eps) * gamma + beta


def _tc1_body(x_ref, wg_ref, degp_ref, z1_ref, z1b_ref, dis_ref):
    deg = degp_ref[0, :N] + degp_ref[1, :N] + 1.0
    dis = lax.rsqrt(deg)
    dis_ref[...] = dis
    xl = jnp.dot(x_ref[...], wg_ref[...], preferred_element_type=jnp.float32)
    z1 = xl * dis[:, None]
    z1_ref[...] = z1
    z1b_ref[...] = z1.astype(jnp.bfloat16)


def _tc2_body(acc_ref, z1_ref, dis_ref, bg_ref, g1_ref, be1_ref, wd_ref,
              h_ref, z2_ref, z2b_ref):
    dis = dis_ref[...]
    z1 = z1_ref[...]
    tot = (acc_ref[0, :N, :].astype(jnp.float32)
           + acc_ref[1, :N, :].astype(jnp.float32) + z1)
    h = _bn(tot * dis[:, None] + bg_ref[...], g1_ref[...], be1_ref[...])
    h_ref[...] = h
    xd = jnp.dot(h, wd_ref[...], preferred_element_type=jnp.float32)
    z2 = xd * dis[:, None]
    z2_ref[...] = z2
    z2b_ref[...] = z2.astype(jnp.bfloat16)


def _tc3_body(acc_ref, z2_ref, h_ref, dis_ref, bd_ref, g3_ref, be3_ref,
              wo_ref, bo_ref, out_ref):
    dis = dis_ref[...]
    h = h_ref[...]
    tot = (acc_ref[0, :N, :].astype(jnp.float32)
           + acc_ref[1, :N, :].astype(jnp.float32) + z2_ref[...])
    o = tot * dis[:, None] * h + bd_ref[...]
    o = _bn(o, g3_ref[...], be3_ref[...])
    out_ref[...] = (
        jnp.dot(h, wo_ref[:D, :], preferred_element_type=jnp.float32)
        + jnp.dot(o, wo_ref[D:, :], preferred_element_type=jnp.float32)
        + bo_ref[...]
    )


def kernel(x, W_gcn, b_gcn, g1, be1, W_deg, b_deg, g3, be3, W_out, b_out,
           edge_index):
    row = edge_index[0]
    col = edge_index[1]
    pad = E_PAD - E
    rowp = jnp.concatenate([row, jnp.zeros((pad,), jnp.int32)])
    colp = jnp.concatenate([col, jnp.full((pad,), SINK, jnp.int32)])
    row3d = rowp.reshape(TOT_CH, CHUNK)
    col3d = colp.reshape(TOT_CH, CHUNK)
    zeros_acc = jnp.zeros((NPAD, D), jnp.bfloat16)

    deg_partials = _deg_kernel(col3d)

    z1, z1b, dis = pl.pallas_call(
        _tc1_body,
        out_shape=(
            jax.ShapeDtypeStruct((N, D), jnp.float32),
            jax.ShapeDtypeStruct((N, D), jnp.bfloat16),
            jax.ShapeDtypeStruct((N,), jnp.float32),
        ),
    )(x, W_gcn, deg_partials)

    acc1 = _prop_kernel(z1b, row3d, col3d, zeros_acc)

    h, z2, z2b = pl.pallas_call(
        _tc2_body,
        out_shape=(
            jax.ShapeDtypeStruct((N, D), jnp.float32),
            jax.ShapeDtypeStruct((N, D), jnp.float32),
            jax.ShapeDtypeStruct((N, D), jnp.bfloat16),
        ),
    )(acc1, z1, dis, b_gcn, g1, be1, W_deg)

    acc2 = _prop_kernel(z2b, row3d, col3d, zeros_acc)

    out = pl.pallas_call(
        _tc3_body,
        out_shape=jax.ShapeDtypeStruct((N, D), jnp.float32),
    )(acc2, z2, h, dis, b_deg, g3, be3, W_out, b_out)

    return out
